# baseline (speedup 1.0000x reference)
import functools

import jax
import jax.numpy as jnp
from jax import lax
from jax.experimental import pallas as pl
from jax.experimental.pallas import tpu as pltpu
from jax.experimental.pallas import tpu_sc as plsc

D = 128
C = 64
NBUF = 8
NW = 32
L = 16


def kernel(id, W, b):
    ids_flat = id.reshape(-1)
    b_flat = b.reshape(-1)
    V = b_flat.size
    N = ids_flat.size
    npw = N // NW
    nck = npw // C
    ngrp = nck // NBUF

    mesh = plsc.VectorSubcoreMesh(core_axis_name="c", subcore_axis_name="s")

    @functools.partial(
        pl.kernel,
        mesh=mesh,
        out_type=[
            jax.ShapeDtypeStruct((N, D), jnp.float32),
            jax.ShapeDtypeStruct((N,), jnp.float32),
        ],
        scratch_types=(
            [pltpu.VMEM((npw,), jnp.int32)]
            + [pltpu.VMEM((C, D), jnp.float32)] * NBUF
            + [pltpu.VMEM((C,), jnp.float32)] * NBUF
            + [pltpu.VMEM((L,), jnp.int32),
               pltpu.VMEM((1, D), jnp.float32)]
            + [pltpu.SemaphoreType.DMA] * (2 * NBUF)
        ),
    )
    def k(ids_hbm, w_hbm, b_hbm, out_w, out_b, idx_v, *scr):
        rows = scr[:NBUF]
        bvs = scr[NBUF:2 * NBUF]
        flag_v = scr[2 * NBUF]
        zrow_v = scr[2 * NBUF + 1]
        sg = scr[2 * NBUF + 2:2 * NBUF + 2 + NBUF]
        so = scr[2 * NBUF + 2 + NBUF:]

        cid = lax.axis_index("c")
        sid = lax.axis_index("s")
        wid = sid * 2 + cid
        base = wid * npw

        zi = jnp.zeros((L,), jnp.int32)
        oi = jnp.ones((L,), jnp.int32)
        zf = jnp.zeros((L,), jnp.float32)

        pltpu.sync_copy(ids_hbm.at[pl.ds(base, npw)], idx_v)
        flag_v[...] = zi
        for cc in range(D // L):
            zrow_v[0, pl.ds(cc * L, L)] = zf

        def gath_w(g, t):
            isl = idx_v.at[pl.ds(g * C, C)]
            return pltpu.make_async_copy(w_hbm.at[isl], rows[t], sg[t])

        def gath_b(g, t):
            isl = idx_v.at[pl.ds(g * C, C)]
            return pltpu.make_async_copy(b_hbm.at[isl], bvs[t], sg[t])

        def out_w_cp(g, t):
            return pltpu.make_async_copy(
                rows[t], out_w.at[pl.ds(base + g * C, C)], so[t])

        def out_b_cp(g, t):
            return pltpu.make_async_copy(
                bvs[t], out_b.at[pl.ds(base + g * C, C)], so[t])

        def issue(g, t):
            gath_w(g, t).start()
            gath_b(g, t).start()

        def wait_out(g, t):
            out_w_cp(g, t).wait()
            out_b_cp(g, t).wait()

        def process(g, t):
            gath_w(g, t).wait()
            gath_b(g, t).wait()
            off = g * C
            cnt = flag_v[...]
            for kk in range(C // L):
                idx16 = idx_v[pl.ds(off + kk * L, L)]
                zmask = idx16 == 0
                bv16 = bvs[t][pl.ds(kk * L, L)]
                bvs[t][pl.ds(kk * L, L)] = jnp.where(zmask, zf, bv16)
                cnt = cnt + jnp.where(zmask, oi, zi)
            flag_v[...] = cnt
            out_w_cp(g, t).start()
            out_b_cp(g, t).start()

        H = NBUF // 2
        for g in range(H):
            issue(g, g)
        for g in range(H):
            issue(g + H, g + H)
            process(g, g)

        def body(j, carry):
            for t in range(NBUF):
                g = j * NBUF + H + t
                wait_out(g - H, t)
                issue(g + H, t)
                process(g, (H + t) % NBUF)
            return carry

        lax.fori_loop(0, (nck - NBUF) // NBUF, body, 0)
        for g in range(nck - H, nck):
            wait_out(g - H, (g - H) % NBUF)
            process(g, g % NBUF)
        for g in range(nck - H, nck):
            wait_out(g, g % NBUF)

        cnt = flag_v[...]
        total = cnt[0]
        for l in range(1, L):
            total = total + cnt[l]

        @pl.when(total > 0)
        def _():
            def zbody(g2, carry):
                idx16 = idx_v[pl.ds(g2 * L, L)]
                for l in range(L):
                    sv = idx16[l]

                    @pl.when(sv == 0)
                    def _():
                        row = base + g2 * L + l
                        pltpu.sync_copy(zrow_v, out_w.at[pl.ds(row, 1)])

                return carry

            lax.fori_loop(0, npw // L, zbody, 0)

    w_out, b_out = k(ids_flat, W, b_flat)
    return (w_out.reshape(id.shape[0], id.shape[1], D),
            b_out.reshape(id.shape[0], id.shape[1], 1))


# speedup vs baseline: 1.1799x; 1.1799x over previous
import functools

import jax
import jax.numpy as jnp
from jax import lax
from jax.experimental import pallas as pl
from jax.experimental.pallas import tpu as pltpu
from jax.experimental.pallas import tpu_sc as plsc

D = 128
C = 64
NBUF = 8
NW = 32
L = 16


def kernel(id, W, b):
    ids_flat = id.reshape(-1)
    b_flat = b.reshape(-1)
    V = b_flat.size
    N = ids_flat.size
    npw = N // NW
    nck = npw // C
    ngrp = nck // NBUF

    mesh = plsc.VectorSubcoreMesh(core_axis_name="c", subcore_axis_name="s")

    @functools.partial(
        pl.kernel,
        mesh=mesh,
        out_type=[
            jax.ShapeDtypeStruct((N, D), jnp.float32),
            jax.ShapeDtypeStruct((N,), jnp.float32),
        ],
        scratch_types=(
            [pltpu.VMEM((npw,), jnp.int32)]
            + [pltpu.VMEM((C, D), jnp.float32)] * NBUF
            + [pltpu.VMEM((C,), jnp.float32)] * NBUF
            + [pltpu.VMEM((nck * L,), jnp.int32),
               pltpu.VMEM((1, D), jnp.float32)]
            + [pltpu.SemaphoreType.DMA] * (2 * NBUF)
        ),
    )
    def k(ids_hbm, w_hbm, b_hbm, out_w, out_b, idx_v, *scr):
        rows = scr[:NBUF]
        bvs = scr[NBUF:2 * NBUF]
        flags_v = scr[2 * NBUF]
        zrow_v = scr[2 * NBUF + 1]
        sg = scr[2 * NBUF + 2:2 * NBUF + 2 + NBUF]
        so = scr[2 * NBUF + 2 + NBUF:]

        cid = lax.axis_index("c")
        sid = lax.axis_index("s")
        wid = sid * 2 + cid
        base = wid * npw

        zi = jnp.zeros((L,), jnp.int32)
        oi = jnp.ones((L,), jnp.int32)
        zf = jnp.zeros((L,), jnp.float32)

        pltpu.sync_copy(ids_hbm.at[pl.ds(base, npw)], idx_v)
        for cc in range(D // L):
            zrow_v[0, pl.ds(cc * L, L)] = zf

        def gath_w(g, t):
            isl = idx_v.at[pl.ds(g * C, C)]
            return pltpu.make_async_copy(w_hbm.at[isl], rows[t], sg[t])

        def gath_b(g, t):
            isl = idx_v.at[pl.ds(g * C, C)]
            return pltpu.make_async_copy(b_hbm.at[isl], bvs[t], sg[t])

        def out_w_cp(g, t):
            return pltpu.make_async_copy(
                rows[t], out_w.at[pl.ds(base + g * C, C)], so[t])

        def out_b_cp(g, t):
            return pltpu.make_async_copy(
                bvs[t], out_b.at[pl.ds(base + g * C, C)], so[t])

        def issue(g, t):
            gath_w(g, t).start()
            gath_b(g, t).start()

        def wait_out(g, t):
            out_w_cp(g, t).wait()
            out_b_cp(g, t).wait()

        def process(g, t):
            gath_w(g, t).wait()
            gath_b(g, t).wait()
            off = g * C
            delta = zi
            for kk in range(C // L):
                idx16 = idx_v[pl.ds(off + kk * L, L)]
                zmask = idx16 == 0
                bv16 = bvs[t][pl.ds(kk * L, L)]
                bvs[t][pl.ds(kk * L, L)] = jnp.where(zmask, zf, bv16)
                delta = delta + jnp.where(zmask, oi, zi)
            flags_v[pl.ds(g * L, L)] = delta
            out_w_cp(g, t).start()
            out_b_cp(g, t).start()

        H = NBUF // 2
        for g in range(H):
            issue(g, g)
        for g in range(H):
            issue(g + H, g + H)
            process(g, g)

        def body(j, carry):
            for t in range(NBUF):
                g = j * NBUF + H + t
                wait_out(g - H, t)
                issue(g + H, t)
                process(g, (H + t) % NBUF)
            return carry

        lax.fori_loop(0, (nck - NBUF) // NBUF, body, 0)
        for g in range(nck - H, nck):
            wait_out(g - H, (g - H) % NBUF)
            process(g, g % NBUF)
        for g in range(nck - H, nck):
            wait_out(g, g % NBUF)

        def zchunk(g, carry):
            delta = flags_v[pl.ds(g * L, L)]
            tz = delta[0]
            for l in range(1, L):
                tz = tz + delta[l]

            @pl.when(tz > 0)
            def _():
                for kk in range(C // L):
                    idx16 = idx_v[pl.ds(g * C + kk * L, L)]
                    for l in range(L):
                        sv = idx16[l]

                        @pl.when(sv == 0)
                        def _():
                            row = base + g * C + kk * L + l
                            pltpu.sync_copy(zrow_v,
                                            out_w.at[pl.ds(row, 1)])

            return carry

        lax.fori_loop(0, nck, zchunk, 0)

    w_out, b_out = k(ids_flat, W, b_flat)
    return (w_out.reshape(id.shape[0], id.shape[1], D),
            b_out.reshape(id.shape[0], id.shape[1], 1))
